# baseline (speedup 1.0000x reference)
import functools

import jax
import jax.numpy as jnp
from jax.experimental import pallas as pl
from jax.experimental.pallas import tpu as pltpu

N = 10000
E = 320000
H = 128
HH = 64
DE = 16
ER = 8000


NR = 2000


def _tables_body(emb_flag, h_ref, Wemb_ref, WA_ref, bA_ref, WB_ref, bB_ref,
                 WD_ref, bD_ref, WE_ref, bE_ref, h0_ref, A_ref, DB_ref, Et_ref):
    if emb_flag:
        h0 = jnp.dot(h_ref[...], Wemb_ref[...], preferred_element_type=jnp.float32)
    else:
        h0 = h_ref[...]
    h0_ref[...] = h0
    A_ref[...] = jnp.dot(h0, WA_ref[...], preferred_element_type=jnp.float32) + bA_ref[...]
    D = jnp.dot(h0, WD_ref[...], preferred_element_type=jnp.float32) + bD_ref[...]
    B = jnp.dot(h0, WB_ref[...], preferred_element_type=jnp.float32) + bB_ref[...]
    Ev = jnp.dot(h0, WE_ref[...], preferred_element_type=jnp.float32) + bE_ref[...]
    DB_ref[0] = jnp.concatenate([D[:, :HH], B[:, :HH]], axis=1)
    DB_ref[1] = jnp.concatenate([D[:, HH:], B[:, HH:]], axis=1)
    Et_ref[0] = Ev[:, :HH]
    Et_ref[1] = Ev[:, HH:]


def _tc_tables(h, Wemb, WA, bA, WB, bB, WD, bD, WE, bE, emb):
    wspec = pl.BlockSpec((H, H), lambda i: (0, 0))
    bspec = pl.BlockSpec((1, H), lambda i: (0, 0))
    return pl.pallas_call(
        functools.partial(_tables_body, emb),
        grid=(N // NR,),
        in_specs=[
            pl.BlockSpec((NR, H), lambda i: (i, 0)),
            wspec, wspec, bspec, wspec, bspec, wspec, bspec, wspec, bspec,
        ],
        out_specs=(
            pl.BlockSpec((NR, H), lambda i: (i, 0)),
            pl.BlockSpec((NR, H), lambda i: (i, 0)),
            pl.BlockSpec((2, NR, H), lambda i: (0, i, 0)),
            pl.BlockSpec((2, NR, HH), lambda i: (0, i, 0)),
        ),
        out_shape=(
            jax.ShapeDtypeStruct((N, H), jnp.float32),
            jax.ShapeDtypeStruct((N, H), jnp.float32),
            jax.ShapeDtypeStruct((2, N, H), jnp.float32),
            jax.ShapeDtypeStruct((2, N, HH), jnp.float32),
        ),
    )(h, Wemb, WA, bA, WB, bB, WD, bD, WE, bE)


def _edge_pre_body(e_ref, Wembe_ref, WC_ref, bC_ref, C_ref):
    F = jnp.dot(Wembe_ref[...], WC_ref[...], preferred_element_type=jnp.float32)
    c = jnp.dot(1.0 / e_ref[...], F, preferred_element_type=jnp.float32) + bC_ref[...]
    C_ref[0] = c[:, :HH]
    C_ref[1] = c[:, HH:]


def _tc_edge_pre(e, Wembe, WC0, bC0):
    return pl.pallas_call(
        _edge_pre_body,
        grid=(E // ER,),
        in_specs=[
            pl.BlockSpec((ER, DE), lambda i: (i, 0)),
            pl.BlockSpec((DE, H), lambda i: (0, 0)),
            pl.BlockSpec((H, H), lambda i: (0, 0)),
            pl.BlockSpec((1, H), lambda i: (0, 0)),
        ],
        out_specs=pl.BlockSpec((2, ER, HH), lambda i: (0, i, 0)),
        out_shape=jax.ShapeDtypeStruct((2, E, HH), jnp.float32),
    )(e, Wembe, WC0, bC0)


def _hupdate_body(h0_ref, A0_ref, num_ref, den_ref, gh_ref, bh_ref,
                  stats_ref, h1_ref, muvar_ref):
    num = jnp.concatenate([num_ref[0], num_ref[1]], axis=1)
    den = jnp.concatenate([den_ref[0], den_ref[1]], axis=1)
    h_new = A0_ref[...] + num / (den + 1e-6)
    mu = jnp.mean(h_new, axis=0, keepdims=True)
    var = jnp.mean((h_new - mu) ** 2, axis=0, keepdims=True)
    hbn = gh_ref[...] * (h_new - mu) / jnp.sqrt(var + 1e-5) + bh_ref[...]
    h1_ref[...] = h0_ref[...] + jnp.maximum(hbn, 0.0)
    s0 = jnp.sum(stats_ref[0], axis=0)
    s1 = jnp.sum(stats_ref[1], axis=0)
    esum = jnp.concatenate([s0[:HH], s1[:HH]])[None, :]
    esq = jnp.concatenate([s0[HH:], s1[HH:]])[None, :]
    emu = esum / E
    evar = esq / E - emu * emu
    muvar_ref[...] = jnp.concatenate([emu, evar], axis=0)


def _tc_hupdate(h0, A0, num, den, gh, bh, stats):
    return pl.pallas_call(
        _hupdate_body,
        out_shape=(
            jax.ShapeDtypeStruct((N, H), jnp.float32),
            jax.ShapeDtypeStruct((2, H), jnp.float32),
        ),
    )(h0, A0, num, den, gh, bh, stats)


def _edge_mid_body(e_ref, en0_ref, muvar_ref, ge_ref, be_ref, Wembe_ref,
                   WC_ref, bC_ref, P_ref):
    en = jnp.concatenate([en0_ref[0], en0_ref[1]], axis=1)
    mu = muvar_ref[0][None, :]
    var = muvar_ref[1][None, :]
    t = ge_ref[...] * (en - mu) / jnp.sqrt(var + 1e-5) + be_ref[...]
    t = jnp.maximum(t, 0.0)
    F = jnp.dot(Wembe_ref[...], WC_ref[...], preferred_element_type=jnp.float32)
    p = (jnp.dot(t, WC_ref[...], preferred_element_type=jnp.float32)
         + jnp.dot(1.0 / e_ref[...], F, preferred_element_type=jnp.float32)
         + bC_ref[...])
    P_ref[0] = p[:, :HH]
    P_ref[1] = p[:, HH:]


def _tc_edge_mid(e, en0, muvar, ge, be, Wembe, WC1, bC1):
    return pl.pallas_call(
        _edge_mid_body,
        grid=(E // ER,),
        in_specs=[
            pl.BlockSpec((ER, DE), lambda i: (i, 0)),
            pl.BlockSpec((2, ER, HH), lambda i: (0, i, 0)),
            pl.BlockSpec((2, H), lambda i: (0, 0)),
            pl.BlockSpec((1, H), lambda i: (0, 0)),
            pl.BlockSpec((1, H), lambda i: (0, 0)),
            pl.BlockSpec((DE, H), lambda i: (0, 0)),
            pl.BlockSpec((H, H), lambda i: (0, 0)),
            pl.BlockSpec((1, H), lambda i: (0, 0)),
        ],
        out_specs=pl.BlockSpec((2, ER, HH), lambda i: (0, i, 0)),
        out_shape=jax.ShapeDtypeStruct((2, E, HH), jnp.float32),
    )(e, en0, muvar, ge, be, Wembe, WC1, bC1)


def _final_body(h1_ref, A1_ref, num_ref, den_ref, gh_ref, bh_ref, state_ref,
                action_ref, W1_ref, b1_ref, W2_ref, b2_ref, W3_ref, b3_ref,
                out_ref):
    num = jnp.concatenate([num_ref[0], num_ref[1]], axis=1)
    den = jnp.concatenate([den_ref[0], den_ref[1]], axis=1)
    h_new = A1_ref[...] + num / (den + 1e-6)
    mu = jnp.mean(h_new, axis=0, keepdims=True)
    var = jnp.mean((h_new - mu) ** 2, axis=0, keepdims=True)
    hbn = gh_ref[...] * (h_new - mu) / jnp.sqrt(var + 1e-5) + bh_ref[...]
    h2 = h1_ref[...] + jnp.maximum(hbn, 0.0)
    hg = jnp.mean(h2, axis=0, keepdims=True)
    z = jnp.concatenate([hg, state_ref[...], action_ref[...]], axis=1)
    z = jnp.maximum(jnp.dot(z, W1_ref[...], preferred_element_type=jnp.float32) + b1_ref[...], 0.0)
    z = jnp.maximum(jnp.dot(z, W2_ref[...], preferred_element_type=jnp.float32) + b2_ref[...], 0.0)
    out_ref[...] = jnp.dot(z, W3_ref[...], preferred_element_type=jnp.float32) + b3_ref[...]


def _tc_final(h1, A1, num, den, gh, bh, state, action, W1, b1, W2, b2, W3, b3):
    return pl.pallas_call(
        _final_body,
        out_shape=jax.ShapeDtypeStruct((1, 1), jnp.float32),
    )(h1, A1, num, den, gh, bh, state, action, W1, b1, W2, b2, W3, b3)


def _edge_stage_jnp(C, DB, Et, src, dst, want_en):
    Cf = jnp.concatenate([C[0], C[1]], axis=1)
    D = jnp.concatenate([DB[0][:, :HH], DB[1][:, :HH]], axis=1)
    B = jnp.concatenate([DB[0][:, HH:], DB[1][:, HH:]], axis=1)
    Ev = jnp.concatenate([Et[0], Et[1]], axis=1)
    en = D[src] + Ev[dst] + Cf
    sg = jax.nn.sigmoid(en)
    numf = jax.ops.segment_sum(sg * B[src], dst, num_segments=N)
    denf = jax.ops.segment_sum(sg, dst, num_segments=N)
    num = jnp.stack([numf[:, :HH], numf[:, HH:]])
    den = jnp.stack([denf[:, :HH], denf[:, HH:]])
    if not want_en:
        return num, den, None, None
    ssum = jnp.sum(en, axis=0)
    ssq = jnp.sum(en * en, axis=0)
    stats = jnp.zeros((2, 16, 128), jnp.float32)
    stats = stats.at[0, 0, :HH].set(ssum[:HH]).at[0, 0, HH:].set(ssq[:HH])
    stats = stats.at[1, 0, :HH].set(ssum[HH:]).at[1, 0, HH:].set(ssq[HH:])
    e_new = jnp.stack([en[:, :HH], en[:, HH:]])
    return num, den, e_new, stats


def kernel(h, e, edge_index, state, action, W_emb_h, W_emb_e, WA, bA, WB, bB,
           WC, bC, WD, bD, WE, bE, gh, bh, ge, be, W1, b1, W2, b2, W3, b3):
    src = edge_index[0]
    dst = edge_index[1]
    r = lambda v: v.reshape(1, -1)

    h0, A0, DB0, Et0 = _tc_tables(h, W_emb_h, WA[0], r(bA[0]), WB[0], r(bB[0]),
                                  WD[0], r(bD[0]), WE[0], r(bE[0]), True)
    C0 = _tc_edge_pre(e, W_emb_e, WC[0], r(bC[0]))
    num0, den0, en0, stats0 = _edge_stage_jnp(C0, DB0, Et0, src, dst, True)

    h1, muvar0 = _tc_hupdate(h0, A0, num0, den0, r(gh[0]), r(bh[0]), stats0)
    _, A1, DB1, Et1 = _tc_tables(h1, W_emb_h, WA[1], r(bA[1]), WB[1], r(bB[1]),
                                 WD[1], r(bD[1]), WE[1], r(bE[1]), False)
    P1 = _tc_edge_mid(e, en0, muvar0, r(ge[0]), r(be[0]), W_emb_e, WC[1], r(bC[1]))

    num1, den1, _, _ = _edge_stage_jnp(P1, DB1, Et1, src, dst, False)

    return _tc_final(h1, A1, num1, den1, r(gh[1]), r(bh[1]), state, action,
                     W1, r(b1), W2, r(b2), W3, r(b3))


# speedup vs baseline: 1.4245x; 1.4245x over previous
import functools

import jax
import jax.numpy as jnp
from jax import lax
from jax.experimental import pallas as pl
from jax.experimental.pallas import tpu as pltpu
from jax.experimental.pallas import tpu_sc as plsc

N = 10000
E = 320000
H = 128
HH = 64
DE = 16
ER = 8000
NR = 2000


def _tables_body(emb_flag, h_ref, Wemb_ref, WA_ref, bA_ref, WB_ref, bB_ref,
                 WD_ref, bD_ref, WE_ref, bE_ref, h0_ref, A_ref, DB_ref, Et_ref):
    if emb_flag:
        h0 = jnp.dot(h_ref[...], Wemb_ref[...], preferred_element_type=jnp.float32)
    else:
        h0 = h_ref[...]
    h0_ref[...] = h0
    A_ref[...] = jnp.dot(h0, WA_ref[...], preferred_element_type=jnp.float32) + bA_ref[...]
    D = jnp.dot(h0, WD_ref[...], preferred_element_type=jnp.float32) + bD_ref[...]
    B = jnp.dot(h0, WB_ref[...], preferred_element_type=jnp.float32) + bB_ref[...]
    Ev = jnp.dot(h0, WE_ref[...], preferred_element_type=jnp.float32) + bE_ref[...]
    DB_ref[0] = jnp.concatenate([D[:, :HH], B[:, :HH]], axis=1)
    DB_ref[1] = jnp.concatenate([D[:, HH:], B[:, HH:]], axis=1)
    Et_ref[...] = Ev


def _tc_tables(h, Wemb, WA, bA, WB, bB, WD, bD, WE, bE, emb):
    wspec = pl.BlockSpec((H, H), lambda i: (0, 0))
    bspec = pl.BlockSpec((1, H), lambda i: (0, 0))
    return pl.pallas_call(
        functools.partial(_tables_body, emb),
        grid=(N // NR,),
        in_specs=[
            pl.BlockSpec((NR, H), lambda i: (i, 0)),
            wspec, wspec, bspec, wspec, bspec, wspec, bspec, wspec, bspec,
        ],
        out_specs=(
            pl.BlockSpec((NR, H), lambda i: (i, 0)),
            pl.BlockSpec((NR, H), lambda i: (i, 0)),
            pl.BlockSpec((2, NR, H), lambda i: (0, i, 0)),
            pl.BlockSpec((NR, H), lambda i: (i, 0)),
        ),
        out_shape=(
            jax.ShapeDtypeStruct((N, H), jnp.float32),
            jax.ShapeDtypeStruct((N, H), jnp.float32),
            jax.ShapeDtypeStruct((2, N, H), jnp.float32),
            jax.ShapeDtypeStruct((N, H), jnp.float32),
        ),
    )(h, Wemb, WA, bA, WB, bB, WD, bD, WE, bE)


def _edge_pre_body(e_ref, Wembe_ref, WC_ref, bC_ref, C_ref):
    F = jnp.dot(Wembe_ref[...], WC_ref[...], preferred_element_type=jnp.float32)
    c = jnp.dot(1.0 / e_ref[...], F, preferred_element_type=jnp.float32) + bC_ref[...]
    C_ref[0] = c[:, :HH]
    C_ref[1] = c[:, HH:]


def _tc_edge_pre(e, Wembe, WC0, bC0):
    return pl.pallas_call(
        _edge_pre_body,
        grid=(E // ER,),
        in_specs=[
            pl.BlockSpec((ER, DE), lambda i: (i, 0)),
            pl.BlockSpec((DE, H), lambda i: (0, 0)),
            pl.BlockSpec((H, H), lambda i: (0, 0)),
            pl.BlockSpec((1, H), lambda i: (0, 0)),
        ],
        out_specs=pl.BlockSpec((2, ER, HH), lambda i: (0, i, 0)),
        out_shape=jax.ShapeDtypeStruct((2, E, HH), jnp.float32),
    )(e, Wembe, WC0, bC0)


NSC = 2
NW = 16
CH = 80
EPT = E // NW
NCH = EPT // CH
NZ = 624
NZT = N - NZ * NW

_SC_MESH = plsc.VectorSubcoreMesh(core_axis_name="c", subcore_axis_name="s",
                                  num_cores=NSC, num_subcores=NW)


def _sc_edge_body(want_en, want_stats, srcadj, dstraw, C, DB, Et,
                  nd_hbm, en_hbm, stats_hbm, isrc, idst, cbuf, dbbuf, ebuf,
                  vbuf, statsbuf, acc, sem1, sem2):
    c = lax.axis_index("c")
    s = lax.axis_index("s")

    def zrow(rr, carry):
        for q in range(8):
            vbuf[rr, pl.ds(q * 16, 16)] = jnp.zeros((16,), jnp.float32)
        return carry

    lax.fori_loop(0, CH, zrow, 0)
    nfull = NZ // CH
    for k in range(nfull):
        pltpu.sync_copy(vbuf, acc.at[pl.ds(s * NZ + k * CH, CH)])
    rem = NZ - nfull * CH
    if rem:
        pltpu.sync_copy(vbuf.at[pl.ds(0, rem)], acc.at[pl.ds(s * NZ + nfull * CH, rem)])

    @pl.when(s == 0)
    def _zero_tail():
        pltpu.sync_copy(vbuf.at[pl.ds(0, NZT)], acc.at[pl.ds(NZ * NW, NZT)])

    if want_stats:
        for q in range(8):
            statsbuf[q, :] = jnp.zeros((16,), jnp.float32)
    plsc.subcore_barrier()

    def chunk(j, carry):
        base = s * EPT + j * CH
        pltpu.sync_copy(srcadj.at[pl.ds(c * E + base, CH)], isrc)
        pltpu.sync_copy(dstraw.at[pl.ds(base, CH)], idst)
        pltpu.sync_copy(C.at[pl.ds(c * E + base, CH)], cbuf)
        g1 = pltpu.async_copy(DB.at[isrc], dbbuf, sem1)
        g2 = pltpu.async_copy(Et.at[idst], ebuf, sem2)
        g1.wait()
        g2.wait()

        def row(rr, carry2):
            for q in range(4):
                ql = pl.ds(q * 16, 16)
                x = (cbuf[rr, ql] + dbbuf[rr, ql]
                     + ebuf[rr, pl.ds(c * HH + q * 16, 16)])
                sg = 1.0 / (1.0 + jnp.exp(-x))
                vbuf[rr, ql] = sg * dbbuf[rr, pl.ds(HH + q * 16, 16)]
                vbuf[rr, pl.ds(HH + q * 16, 16)] = sg
                if want_en:
                    cbuf[rr, ql] = x
                if want_stats:
                    statsbuf[q, :] = statsbuf[q, :] + x
                    statsbuf[4 + q, :] = statsbuf[4 + q, :] + x * x
            return carry2

        lax.fori_loop(0, CH, row, 0)
        if want_en:
            pltpu.sync_copy(cbuf, en_hbm.at[pl.ds(c * E + base, CH)])
        pltpu.sync_copy(vbuf, acc.at[idst], add=True)
        return carry

    lax.fori_loop(0, NCH, chunk, 0)

    if want_stats:
        pltpu.sync_copy(statsbuf, stats_hbm.at[c, s])
    plsc.subcore_barrier()
    pltpu.sync_copy(acc.at[pl.ds(s * NZ, NZ)], nd_hbm.at[c, pl.ds(s * NZ, NZ)])

    @pl.when(s == 0)
    def _flush_tail():
        pltpu.sync_copy(acc.at[pl.ds(NZ * NW, NZT)], nd_hbm.at[c, pl.ds(NZ * NW, NZT)])


def _make_sc_edge(want_en, want_stats=None):
    if want_stats is None:
        want_stats = want_en
    outs = [
        jax.ShapeDtypeStruct((2, N, H), jnp.float32),
        jax.ShapeDtypeStruct((2 * E, HH) if want_en else (8, HH), jnp.float32),
        jax.ShapeDtypeStruct((2, NW, 8, 16), jnp.float32),
    ]
    return pl.kernel(
        functools.partial(_sc_edge_body, want_en, want_stats),
        out_type=tuple(outs),
        mesh=_SC_MESH,
        scratch_types=[
            pltpu.VMEM((CH,), jnp.int32),
            pltpu.VMEM((CH,), jnp.int32),
            pltpu.VMEM((CH, HH), jnp.float32),
            pltpu.VMEM((CH, H), jnp.float32),
            pltpu.VMEM((CH, H), jnp.float32),
            pltpu.VMEM((CH, H), jnp.float32),
            pltpu.VMEM((8, 16), jnp.float32),
            pltpu.VMEM_SHARED((N, H), jnp.float32),
            pltpu.SemaphoreType.DMA,
            pltpu.SemaphoreType.DMA,
        ],
    )


_sc_edge0 = _make_sc_edge(True, True)
_sc_edge1 = _make_sc_edge(False, False)


def _edge_stage_sc(C, DB, Et, srcadj, dstraw, want_en):
    fn = _sc_edge0 if want_en else _sc_edge1
    nd, en, stats = fn(srcadj, dstraw, C.reshape(2 * E, HH),
                       DB.reshape(2 * N, H), Et)
    if not want_en:
        return nd, None, None
    return nd, en.reshape(2, E, HH), stats.reshape(2, NW, 128)


def _hupdate_body(h0_ref, A0_ref, nd_ref, gh_ref, bh_ref, stats_ref,
                  h1_ref, muvar_ref):
    num = jnp.concatenate([nd_ref[0][:, :HH], nd_ref[1][:, :HH]], axis=1)
    den = jnp.concatenate([nd_ref[0][:, HH:], nd_ref[1][:, HH:]], axis=1)
    h_new = A0_ref[...] + num / (den + 1e-6)
    mu = jnp.mean(h_new, axis=0, keepdims=True)
    var = jnp.mean((h_new - mu) ** 2, axis=0, keepdims=True)
    hbn = gh_ref[...] * (h_new - mu) / jnp.sqrt(var + 1e-5) + bh_ref[...]
    h1_ref[...] = h0_ref[...] + jnp.maximum(hbn, 0.0)
    s0 = jnp.sum(stats_ref[0], axis=0)
    s1 = jnp.sum(stats_ref[1], axis=0)
    esum = jnp.concatenate([s0[:HH], s1[:HH]])[None, :]
    esq = jnp.concatenate([s0[HH:], s1[HH:]])[None, :]
    emu = esum / E
    evar = esq / E - emu * emu
    muvar_ref[...] = jnp.concatenate([emu, evar], axis=0)


def _tc_hupdate(h0, A0, nd, gh, bh, stats):
    return pl.pallas_call(
        _hupdate_body,
        out_shape=(
            jax.ShapeDtypeStruct((N, H), jnp.float32),
            jax.ShapeDtypeStruct((2, H), jnp.float32),
        ),
    )(h0, A0, nd, gh, bh, stats)


def _edge_mid_body(e_ref, en0_ref, muvar_ref, ge_ref, be_ref, Wembe_ref,
                   WC_ref, bC_ref, P_ref):
    en = jnp.concatenate([en0_ref[0], en0_ref[1]], axis=1)
    mu = muvar_ref[0][None, :]
    var = muvar_ref[1][None, :]
    t = ge_ref[...] * (en - mu) / jnp.sqrt(var + 1e-5) + be_ref[...]
    t = jnp.maximum(t, 0.0)
    F = jnp.dot(Wembe_ref[...], WC_ref[...], preferred_element_type=jnp.float32)
    p = (jnp.dot(t, WC_ref[...], preferred_element_type=jnp.float32)
         + jnp.dot(1.0 / e_ref[...], F, preferred_element_type=jnp.float32)
         + bC_ref[...])
    P_ref[0] = p[:, :HH]
    P_ref[1] = p[:, HH:]


def _tc_edge_mid(e, en0, muvar, ge, be, Wembe, WC1, bC1):
    return pl.pallas_call(
        _edge_mid_body,
        grid=(E // ER,),
        in_specs=[
            pl.BlockSpec((ER, DE), lambda i: (i, 0)),
            pl.BlockSpec((2, ER, HH), lambda i: (0, i, 0)),
            pl.BlockSpec((2, H), lambda i: (0, 0)),
            pl.BlockSpec((1, H), lambda i: (0, 0)),
            pl.BlockSpec((1, H), lambda i: (0, 0)),
            pl.BlockSpec((DE, H), lambda i: (0, 0)),
            pl.BlockSpec((H, H), lambda i: (0, 0)),
            pl.BlockSpec((1, H), lambda i: (0, 0)),
        ],
        out_specs=pl.BlockSpec((2, ER, HH), lambda i: (0, i, 0)),
        out_shape=jax.ShapeDtypeStruct((2, E, HH), jnp.float32),
    )(e, en0, muvar, ge, be, Wembe, WC1, bC1)


def _final_body(h1_ref, A1_ref, nd_ref, gh_ref, bh_ref, state_ref,
                action_ref, W1_ref, b1_ref, W2_ref, b2_ref, W3_ref, b3_ref,
                out_ref):
    num = jnp.concatenate([nd_ref[0][:, :HH], nd_ref[1][:, :HH]], axis=1)
    den = jnp.concatenate([nd_ref[0][:, HH:], nd_ref[1][:, HH:]], axis=1)
    h_new = A1_ref[...] + num / (den + 1e-6)
    mu = jnp.mean(h_new, axis=0, keepdims=True)
    var = jnp.mean((h_new - mu) ** 2, axis=0, keepdims=True)
    hbn = gh_ref[...] * (h_new - mu) / jnp.sqrt(var + 1e-5) + bh_ref[...]
    h2 = h1_ref[...] + jnp.maximum(hbn, 0.0)
    hg = jnp.mean(h2, axis=0, keepdims=True)
    z = jnp.concatenate([hg, state_ref[...], action_ref[...]], axis=1)
    z = jnp.maximum(jnp.dot(z, W1_ref[...], preferred_element_type=jnp.float32) + b1_ref[...], 0.0)
    z = jnp.maximum(jnp.dot(z, W2_ref[...], preferred_element_type=jnp.float32) + b2_ref[...], 0.0)
    out_ref[...] = jnp.dot(z, W3_ref[...], preferred_element_type=jnp.float32) + b3_ref[...]


def _tc_final(h1, A1, nd, gh, bh, state, action, W1, b1, W2, b2, W3, b3):
    return pl.pallas_call(
        _final_body,
        out_shape=jax.ShapeDtypeStruct((1, 1), jnp.float32),
    )(h1, A1, nd, gh, bh, state, action, W1, b1, W2, b2, W3, b3)


def kernel(h, e, edge_index, state, action, W_emb_h, W_emb_e, WA, bA, WB, bB,
           WC, bC, WD, bD, WE, bE, gh, bh, ge, be, W1, b1, W2, b2, W3, b3):
    r = lambda v: v.reshape(1, -1)

    src = edge_index[0]
    dst = edge_index[1]
    srcadj = jnp.concatenate([src, src + N])

    h0, A0, DB0, Et0 = _tc_tables(h, W_emb_h, WA[0], r(bA[0]), WB[0], r(bB[0]),
                                  WD[0], r(bD[0]), WE[0], r(bE[0]), True)
    C0 = _tc_edge_pre(e, W_emb_e, WC[0], r(bC[0]))
    nd0, en0, stats0 = _edge_stage_sc(C0, DB0, Et0, srcadj, dst, True)

    h1, muvar0 = _tc_hupdate(h0, A0, nd0, r(gh[0]), r(bh[0]), stats0)
    _, A1, DB1, Et1 = _tc_tables(h1, W_emb_h, WA[1], r(bA[1]), WB[1], r(bB[1]),
                                 WD[1], r(bD[1]), WE[1], r(bE[1]), False)
    P1 = _tc_edge_mid(e, en0, muvar0, r(ge[0]), r(be[0]), W_emb_e, WC[1], r(bC[1]))

    nd1, _, _ = _edge_stage_sc(P1, DB1, Et1, srcadj, dst, False)

    return _tc_final(h1, A1, nd1, r(gh[1]), r(bh[1]), state, action,
                     W1, r(b1), W2, r(b2), W3, r(b3))


# speedup vs baseline: 1.4981x; 1.0517x over previous
import functools

import jax
import jax.numpy as jnp
from jax import lax
from jax.experimental import pallas as pl
from jax.experimental.pallas import tpu as pltpu
from jax.experimental.pallas import tpu_sc as plsc

N = 10000
E = 320000
H = 128
HH = 64
DE = 16
ER = 8000
NR = 2000


def _tables_body(emb_flag, h_ref, Wemb_ref, WA_ref, bA_ref, WB_ref, bB_ref,
                 WD_ref, bD_ref, WE_ref, bE_ref, h0_ref, A_ref, DB_ref, Et_ref):
    if emb_flag:
        h0 = jnp.dot(h_ref[...], Wemb_ref[...], preferred_element_type=jnp.float32)
    else:
        h0 = h_ref[...]
    h0_ref[...] = h0
    A_ref[...] = jnp.dot(h0, WA_ref[...], preferred_element_type=jnp.float32) + bA_ref[...]
    D = jnp.dot(h0, WD_ref[...], preferred_element_type=jnp.float32) + bD_ref[...]
    B = jnp.dot(h0, WB_ref[...], preferred_element_type=jnp.float32) + bB_ref[...]
    Ev = jnp.dot(h0, WE_ref[...], preferred_element_type=jnp.float32) + bE_ref[...]
    DB_ref[0] = jnp.concatenate([D[:, :HH], B[:, :HH]], axis=1)
    DB_ref[1] = jnp.concatenate([D[:, HH:], B[:, HH:]], axis=1)
    Et_ref[...] = Ev


def _tc_tables(h, Wemb, WA, bA, WB, bB, WD, bD, WE, bE, emb):
    wspec = pl.BlockSpec((H, H), lambda i: (0, 0))
    bspec = pl.BlockSpec((1, H), lambda i: (0, 0))
    return pl.pallas_call(
        functools.partial(_tables_body, emb),
        grid=(N // NR,),
        in_specs=[
            pl.BlockSpec((NR, H), lambda i: (i, 0)),
            wspec, wspec, bspec, wspec, bspec, wspec, bspec, wspec, bspec,
        ],
        out_specs=(
            pl.BlockSpec((NR, H), lambda i: (i, 0)),
            pl.BlockSpec((NR, H), lambda i: (i, 0)),
            pl.BlockSpec((2, NR, H), lambda i: (0, i, 0)),
            pl.BlockSpec((NR, H), lambda i: (i, 0)),
        ),
        out_shape=(
            jax.ShapeDtypeStruct((N, H), jnp.float32),
            jax.ShapeDtypeStruct((N, H), jnp.float32),
            jax.ShapeDtypeStruct((2, N, H), jnp.float32),
            jax.ShapeDtypeStruct((N, H), jnp.float32),
        ),
    )(h, Wemb, WA, bA, WB, bB, WD, bD, WE, bE)


def _edge_pre_body(e_ref, Wembe_ref, WC_ref, bC_ref, C_ref):
    F = jnp.dot(Wembe_ref[...], WC_ref[...], preferred_element_type=jnp.float32)
    c = jnp.dot(1.0 / e_ref[...], F, preferred_element_type=jnp.float32) + bC_ref[...]
    C_ref[0] = c[:, :HH]
    C_ref[1] = c[:, HH:]


def _tc_edge_pre(e, Wembe, WC0, bC0):
    return pl.pallas_call(
        _edge_pre_body,
        grid=(E // ER,),
        in_specs=[
            pl.BlockSpec((ER, DE), lambda i: (i, 0)),
            pl.BlockSpec((DE, H), lambda i: (0, 0)),
            pl.BlockSpec((H, H), lambda i: (0, 0)),
            pl.BlockSpec((1, H), lambda i: (0, 0)),
        ],
        out_specs=pl.BlockSpec((2, ER, HH), lambda i: (0, i, 0)),
        out_shape=jax.ShapeDtypeStruct((2, E, HH), jnp.float32),
    )(e, Wembe, WC0, bC0)


NSC = 2
NW = 16
CH = 40
EPT = E // NW
NCH = EPT // CH
NPAIR = NCH // 2
NZ = 624
NZT = N - NZ * NW

_SC_MESH = plsc.VectorSubcoreMesh(core_axis_name="c", subcore_axis_name="s",
                                  num_cores=NSC, num_subcores=NW)


def _sc_edge_body(want_en, srcadj, dstraw, C, DB, Et, nd_hbm, en_hbm,
                  isrc0, isrc1, idst0, idst1, cbuf0, cbuf1, db0, db1,
                  eb0, eb1, vb0, vb1, enb0, enb1,
                  acc, s_is0, s_is1, s_id0, s_id1, s_c0, s_c1, s_db0, s_db1,
                  s_e0, s_e1, s_sc0, s_sc1, s_en0, s_en1):
    c = lax.axis_index("c")
    s = lax.axis_index("s")
    isrc = (isrc0, isrc1)
    idst = (idst0, idst1)
    cbuf = (cbuf0, cbuf1)
    db = (db0, db1)
    eb = (eb0, eb1)
    vb = (vb0, vb1)
    enb = (enb0, enb1)
    s_is = (s_is0, s_is1)
    s_id = (s_id0, s_id1)
    s_c = (s_c0, s_c1)
    s_db = (s_db0, s_db1)
    s_e = (s_e0, s_e1)
    s_sc = (s_sc0, s_sc1)
    s_en = (s_en0, s_en1)
    tb = s * EPT

    def zrow(rr, carry):
        for q in range(8):
            vb0[rr, pl.ds(q * 16, 16)] = jnp.zeros((16,), jnp.float32)
        return carry

    lax.fori_loop(0, CH, zrow, 0)
    nfull = NZ // CH
    for k in range(nfull):
        pltpu.sync_copy(vb0, acc.at[pl.ds(s * NZ + k * CH, CH)])
    rem = NZ - nfull * CH
    if rem:
        pltpu.sync_copy(vb0.at[pl.ds(0, rem)], acc.at[pl.ds(s * NZ + nfull * CH, rem)])

    @pl.when(s == 0)
    def _zero_tail():
        pltpu.sync_copy(vb0.at[pl.ds(0, NZT)], acc.at[pl.ds(NZ * NW, NZT)])

    plsc.subcore_barrier()

    def idx_issue(j, p):
        base = tb + j * CH
        pltpu.async_copy(srcadj.at[pl.ds(c * E + base, CH)], isrc[p], s_is[p])
        pltpu.async_copy(dstraw.at[pl.ds(base, CH)], idst[p], s_id[p])

    def idx_wait(p):
        pltpu.make_async_copy(srcadj.at[pl.ds(0, CH)], isrc[p], s_is[p]).wait()
        pltpu.make_async_copy(dstraw.at[pl.ds(0, CH)], idst[p], s_id[p]).wait()

    def cbuf_issue(j, p):
        base = (c * E + tb + j * CH) * HH
        pltpu.async_copy(C.at[pl.ds(base, CH * HH)], cbuf[p], s_c[p])

    def cbuf_wait(p):
        pltpu.make_async_copy(C.at[pl.ds(0, CH * HH)], cbuf[p], s_c[p]).wait()

    def gathers_issue(p):
        pltpu.async_copy(DB.at[isrc[p]], db[p], s_db[p])
        pltpu.async_copy(Et.at[idst[p]], eb[p], s_e[p])

    def gathers_wait(p):
        pltpu.make_async_copy(DB.at[pl.ds(0, CH)], db[p], s_db[p]).wait()
        pltpu.make_async_copy(Et.at[pl.ds(0, CH)], eb[p], s_e[p]).wait()

    def scatter_issue(p):
        pltpu.async_copy(vb[p], acc.at[idst[p]], s_sc[p], add=True)

    def scatter_wait(p):
        pltpu.make_async_copy(DB.at[pl.ds(0, CH)], vb[p], s_sc[p]).wait()

    def en_issue(j, p):
        base = (c * E + tb + j * CH) * HH
        pltpu.async_copy(enb[p], en_hbm.at[pl.ds(base, CH * HH)], s_en[p])

    def en_wait(p):
        pltpu.make_async_copy(C.at[pl.ds(0, CH * HH)], enb[p], s_en[p]).wait()

    def compute(p):
        def row(rr, carry2):
            rb = rr * HH
            for q in range(4):
                ql = pl.ds(q * 16, 16)
                x = (cbuf[p][pl.ds(rb + q * 16, 16)] + db[p][rr, ql]
                     + eb[p][rr, pl.ds(c * HH + q * 16, 16)])
                sg = 1.0 / (1.0 + jnp.exp(-x))
                vb[p][rr, ql] = sg * db[p][rr, pl.ds(HH + q * 16, 16)]
                vb[p][rr, pl.ds(HH + q * 16, 16)] = sg
                if want_en:
                    enb[p][pl.ds(rb + q * 16, 16)] = x
            return carry2

        lax.fori_loop(0, CH, row, 0, unroll=2)

    idx_issue(0, 0)
    idx_wait(0)
    gathers_issue(0)
    cbuf_issue(0, 0)
    idx_issue(1, 1)
    cbuf_wait(0)
    gathers_wait(0)
    compute(0)
    scatter_issue(0)
    if want_en:
        en_issue(0, 0)
    idx_wait(1)
    gathers_issue(1)
    cbuf_issue(1, 1)
    scatter_wait(0)
    idx_issue(2, 0)
    cbuf_wait(1)
    gathers_wait(1)
    compute(1)
    scatter_issue(1)
    if want_en:
        en_issue(1, 1)
    idx_wait(0)
    gathers_issue(0)
    cbuf_issue(2, 0)

    def pair(j2, carry):
        je = 2 * j2
        jo = je + 1

        scatter_wait(1)
        idx_issue(jo, 1)
        cbuf_wait(0)
        gathers_wait(0)
        if want_en:
            en_wait(0)
        compute(0)
        scatter_issue(0)
        if want_en:
            en_issue(je, 0)
        idx_wait(1)
        gathers_issue(1)
        cbuf_issue(jo, 1)

        @pl.when(j2 < NPAIR - 1)
        def _pref_next():
            scatter_wait(0)
            idx_issue(jo + 1, 0)

        cbuf_wait(1)
        gathers_wait(1)
        if want_en:
            en_wait(1)
        compute(1)
        scatter_issue(1)
        if want_en:
            en_issue(jo, 1)

        @pl.when(j2 < NPAIR - 1)
        def _pref_next2():
            idx_wait(0)
            gathers_issue(0)
            cbuf_issue(jo + 1, 0)

        return carry

    lax.fori_loop(1, NPAIR, pair, 0)

    scatter_wait(0)
    scatter_wait(1)
    if want_en:
        en_wait(0)
        en_wait(1)
    plsc.subcore_barrier()
    pltpu.sync_copy(acc.at[pl.ds(s * NZ, NZ)], nd_hbm.at[c, pl.ds(s * NZ, NZ)])

    @pl.when(s == 0)
    def _flush_tail():
        pltpu.sync_copy(acc.at[pl.ds(NZ * NW, NZT)], nd_hbm.at[c, pl.ds(NZ * NW, NZT)])


def _make_sc_edge(want_en):
    outs = [
        jax.ShapeDtypeStruct((2, N, H), jnp.float32),
        jax.ShapeDtypeStruct((2 * E * HH,) if want_en else (512,), jnp.float32),
    ]
    sems = [pltpu.SemaphoreType.DMA] * 14
    return pl.kernel(
        functools.partial(_sc_edge_body, want_en),
        out_type=tuple(outs),
        mesh=_SC_MESH,
        scratch_types=[
            pltpu.VMEM((CH,), jnp.int32),
            pltpu.VMEM((CH,), jnp.int32),
            pltpu.VMEM((CH,), jnp.int32),
            pltpu.VMEM((CH,), jnp.int32),
            pltpu.VMEM((CH * HH,), jnp.float32),
            pltpu.VMEM((CH * HH,), jnp.float32),
            pltpu.VMEM((CH, H), jnp.float32),
            pltpu.VMEM((CH, H), jnp.float32),
            pltpu.VMEM((CH, H), jnp.float32),
            pltpu.VMEM((CH, H), jnp.float32),
            pltpu.VMEM((CH, H), jnp.float32),
            pltpu.VMEM((CH, H), jnp.float32),
            pltpu.VMEM((CH * HH,), jnp.float32),
            pltpu.VMEM((CH * HH,), jnp.float32),
            pltpu.VMEM_SHARED((N, H), jnp.float32),
        ] + sems,
    )


_sc_edge0 = _make_sc_edge(True)
_sc_edge1 = _make_sc_edge(False)


def _edge_stage_sc(C, DB, Et, srcadj, dstraw, want_en):
    fn = _sc_edge0 if want_en else _sc_edge1
    nd, en = fn(srcadj, dstraw, C.reshape(2 * E * HH),
                DB.reshape(2 * N, H), Et)
    if not want_en:
        return nd, None
    return nd, en.reshape(2, E, HH)


def _stats_body(en_ref, muvar_ref, acc_ref):
    i = pl.program_id(0)

    @pl.when(i == 0)
    def _init():
        acc_ref[...] = jnp.zeros((2, H), jnp.float32)

    blk = en_ref[...]
    s0 = jnp.sum(blk[0], axis=0)
    s1 = jnp.sum(blk[1], axis=0)
    q0 = jnp.sum(blk[0] * blk[0], axis=0)
    q1 = jnp.sum(blk[1] * blk[1], axis=0)
    acc_ref[0, :] += jnp.concatenate([s0, s1])
    acc_ref[1, :] += jnp.concatenate([q0, q1])

    @pl.when(i == (E // ER) - 1)
    def _fin():
        emu = acc_ref[0, :] / E
        muvar_ref[0, :] = emu
        muvar_ref[1, :] = acc_ref[1, :] / E - emu * emu


def _tc_stats(en):
    return pl.pallas_call(
        _stats_body,
        grid=(E // ER,),
        in_specs=[pl.BlockSpec((2, ER, HH), lambda i: (0, i, 0))],
        out_specs=pl.BlockSpec((2, H), lambda i: (0, 0)),
        out_shape=jax.ShapeDtypeStruct((2, H), jnp.float32),
        scratch_shapes=[pltpu.VMEM((2, H), jnp.float32)],
    )(en)


def _hupdate_body(h0_ref, A0_ref, nd_ref, gh_ref, bh_ref, h1_ref):
    num = jnp.concatenate([nd_ref[0][:, :HH], nd_ref[1][:, :HH]], axis=1)
    den = jnp.concatenate([nd_ref[0][:, HH:], nd_ref[1][:, HH:]], axis=1)
    h_new = A0_ref[...] + num / (den + 1e-6)
    mu = jnp.mean(h_new, axis=0, keepdims=True)
    var = jnp.mean((h_new - mu) ** 2, axis=0, keepdims=True)
    hbn = gh_ref[...] * (h_new - mu) / jnp.sqrt(var + 1e-5) + bh_ref[...]
    h1_ref[...] = h0_ref[...] + jnp.maximum(hbn, 0.0)


def _tc_hupdate(h0, A0, nd, gh, bh):
    return pl.pallas_call(
        _hupdate_body,
        out_shape=jax.ShapeDtypeStruct((N, H), jnp.float32),
    )(h0, A0, nd, gh, bh)


def _edge_mid_body(e_ref, en0_ref, muvar_ref, ge_ref, be_ref, Wembe_ref,
                   WC_ref, bC_ref, P_ref):
    en = jnp.concatenate([en0_ref[0], en0_ref[1]], axis=1)
    mu = muvar_ref[0][None, :]
    var = muvar_ref[1][None, :]
    t = ge_ref[...] * (en - mu) / jnp.sqrt(var + 1e-5) + be_ref[...]
    t = jnp.maximum(t, 0.0)
    F = jnp.dot(Wembe_ref[...], WC_ref[...], preferred_element_type=jnp.float32)
    p = (jnp.dot(t, WC_ref[...], preferred_element_type=jnp.float32)
         + jnp.dot(1.0 / e_ref[...], F, preferred_element_type=jnp.float32)
         + bC_ref[...])
    P_ref[0] = p[:, :HH]
    P_ref[1] = p[:, HH:]


def _tc_edge_mid(e, en0, muvar, ge, be, Wembe, WC1, bC1):
    return pl.pallas_call(
        _edge_mid_body,
        grid=(E // ER,),
        in_specs=[
            pl.BlockSpec((ER, DE), lambda i: (i, 0)),
            pl.BlockSpec((2, ER, HH), lambda i: (0, i, 0)),
            pl.BlockSpec((2, H), lambda i: (0, 0)),
            pl.BlockSpec((1, H), lambda i: (0, 0)),
            pl.BlockSpec((1, H), lambda i: (0, 0)),
            pl.BlockSpec((DE, H), lambda i: (0, 0)),
            pl.BlockSpec((H, H), lambda i: (0, 0)),
            pl.BlockSpec((1, H), lambda i: (0, 0)),
        ],
        out_specs=pl.BlockSpec((2, ER, HH), lambda i: (0, i, 0)),
        out_shape=jax.ShapeDtypeStruct((2, E, HH), jnp.float32),
    )(e, en0, muvar, ge, be, Wembe, WC1, bC1)


def _final_body(h1_ref, A1_ref, nd_ref, gh_ref, bh_ref, state_ref,
                action_ref, W1_ref, b1_ref, W2_ref, b2_ref, W3_ref, b3_ref,
                out_ref):
    num = jnp.concatenate([nd_ref[0][:, :HH], nd_ref[1][:, :HH]], axis=1)
    den = jnp.concatenate([nd_ref[0][:, HH:], nd_ref[1][:, HH:]], axis=1)
    h_new = A1_ref[...] + num / (den + 1e-6)
    mu = jnp.mean(h_new, axis=0, keepdims=True)
    var = jnp.mean((h_new - mu) ** 2, axis=0, keepdims=True)
    hbn = gh_ref[...] * (h_new - mu) / jnp.sqrt(var + 1e-5) + bh_ref[...]
    h2 = h1_ref[...] + jnp.maximum(hbn, 0.0)
    hg = jnp.mean(h2, axis=0, keepdims=True)
    z = jnp.concatenate([hg, state_ref[...], action_ref[...]], axis=1)
    z = jnp.maximum(jnp.dot(z, W1_ref[...], preferred_element_type=jnp.float32) + b1_ref[...], 0.0)
    z = jnp.maximum(jnp.dot(z, W2_ref[...], preferred_element_type=jnp.float32) + b2_ref[...], 0.0)
    out_ref[...] = jnp.dot(z, W3_ref[...], preferred_element_type=jnp.float32) + b3_ref[...]


def _tc_final(h1, A1, nd, gh, bh, state, action, W1, b1, W2, b2, W3, b3):
    return pl.pallas_call(
        _final_body,
        out_shape=jax.ShapeDtypeStruct((1, 1), jnp.float32),
    )(h1, A1, nd, gh, bh, state, action, W1, b1, W2, b2, W3, b3)


def kernel(h, e, edge_index, state, action, W_emb_h, W_emb_e, WA, bA, WB, bB,
           WC, bC, WD, bD, WE, bE, gh, bh, ge, be, W1, b1, W2, b2, W3, b3):
    r = lambda v: v.reshape(1, -1)

    src = edge_index[0]
    dst = edge_index[1]
    srcadj = jnp.concatenate([src, src + N])

    h0, A0, DB0, Et0 = _tc_tables(h, W_emb_h, WA[0], r(bA[0]), WB[0], r(bB[0]),
                                  WD[0], r(bD[0]), WE[0], r(bE[0]), True)
    C0 = _tc_edge_pre(e, W_emb_e, WC[0], r(bC[0]))
    nd0, en0 = _edge_stage_sc(C0, DB0, Et0, srcadj, dst, True)

    muvar0 = _tc_stats(en0)
    h1 = _tc_hupdate(h0, A0, nd0, r(gh[0]), r(bh[0]))
    _, A1, DB1, Et1 = _tc_tables(h1, W_emb_h, WA[1], r(bA[1]), WB[1], r(bB[1]),
                                 WD[1], r(bD[1]), WE[1], r(bE[1]), False)
    P1 = _tc_edge_mid(e, en0, muvar0, r(ge[0]), r(be[0]), W_emb_e, WC[1], r(bC[1]))

    nd1, _ = _edge_stage_sc(P1, DB1, Et1, srcadj, dst, False)

    return _tc_final(h1, A1, nd1, r(gh[1]), r(bh[1]), state, action,
                     W1, r(b1), W2, r(b2), W3, r(b3))


# speedup vs baseline: 1.6722x; 1.1162x over previous
import functools

import jax
import jax.numpy as jnp
from jax import lax
from jax.experimental import pallas as pl
from jax.experimental.pallas import tpu as pltpu
from jax.experimental.pallas import tpu_sc as plsc

N = 10000
E = 320000
H = 128
HH = 64
DE = 16
ER = 8000
NR = 2000


def _tables_body(emb_flag, h_ref, Wemb_ref, WA_ref, bA_ref, WB_ref, bB_ref,
                 WD_ref, bD_ref, WE_ref, bE_ref, h0_ref, A_ref, DB_ref, Et_ref):
    if emb_flag:
        h0 = jnp.dot(h_ref[...], Wemb_ref[...], preferred_element_type=jnp.float32)
    else:
        h0 = h_ref[...]
    h0_ref[...] = h0
    A_ref[...] = jnp.dot(h0, WA_ref[...], preferred_element_type=jnp.float32) + bA_ref[...]
    D = jnp.dot(h0, WD_ref[...], preferred_element_type=jnp.float32) + bD_ref[...]
    B = jnp.dot(h0, WB_ref[...], preferred_element_type=jnp.float32) + bB_ref[...]
    Ev = jnp.dot(h0, WE_ref[...], preferred_element_type=jnp.float32) + bE_ref[...]
    DB_ref[0] = jnp.concatenate([D[:, :HH], B[:, :HH]], axis=1)
    DB_ref[1] = jnp.concatenate([D[:, HH:], B[:, HH:]], axis=1)
    Et_ref[...] = Ev


def _tc_tables(h, Wemb, WA, bA, WB, bB, WD, bD, WE, bE, emb):
    wspec = pl.BlockSpec((H, H), lambda i: (0, 0))
    bspec = pl.BlockSpec((1, H), lambda i: (0, 0))
    return pl.pallas_call(
        functools.partial(_tables_body, emb),
        grid=(N // NR,),
        in_specs=[
            pl.BlockSpec((NR, H), lambda i: (i, 0)),
            wspec, wspec, bspec, wspec, bspec, wspec, bspec, wspec, bspec,
        ],
        out_specs=(
            pl.BlockSpec((NR, H), lambda i: (i, 0)),
            pl.BlockSpec((NR, H), lambda i: (i, 0)),
            pl.BlockSpec((2, NR, H), lambda i: (0, i, 0)),
            pl.BlockSpec((NR, H), lambda i: (i, 0)),
        ),
        out_shape=(
            jax.ShapeDtypeStruct((N, H), jnp.float32),
            jax.ShapeDtypeStruct((N, H), jnp.float32),
            jax.ShapeDtypeStruct((2, N, H), jnp.float32),
            jax.ShapeDtypeStruct((N, H), jnp.float32),
        ),
    )(h, Wemb, WA, bA, WB, bB, WD, bD, WE, bE)


def _edge_pre_body(e_ref, Wembe_ref, WC_ref, bC_ref, C_ref):
    F = jnp.dot(Wembe_ref[...], WC_ref[...], preferred_element_type=jnp.float32)
    c = jnp.dot(1.0 / e_ref[...], F, preferred_element_type=jnp.float32) + bC_ref[...]
    C_ref[0] = c[:, :HH]
    C_ref[1] = c[:, HH:]


def _tc_edge_pre(e, Wembe, WC0, bC0):
    return pl.pallas_call(
        _edge_pre_body,
        grid=(E // ER,),
        in_specs=[
            pl.BlockSpec((ER, DE), lambda i: (i, 0)),
            pl.BlockSpec((DE, H), lambda i: (0, 0)),
            pl.BlockSpec((H, H), lambda i: (0, 0)),
            pl.BlockSpec((1, H), lambda i: (0, 0)),
        ],
        out_specs=pl.BlockSpec((2, ER, HH), lambda i: (0, i, 0)),
        out_shape=jax.ShapeDtypeStruct((2, E, HH), jnp.float32),
    )(e, Wembe, WC0, bC0)


NSC = 2
NW = 16
CH = 40
EPT = E // NW
NCH = EPT // CH
NPAIR = NCH // 2
NZ = 624
NZT = N - NZ * NW

_SC_MESH = plsc.VectorSubcoreMesh(core_axis_name="c", subcore_axis_name="s",
                                  num_cores=NSC, num_subcores=NW)


def _sc_edge_body(want_en, srcadj, dstraw, C, DB, Et, nd_hbm, en_hbm,
                  isrc0, isrc1, idst0, idst1, cbuf0, cbuf1, db0, db1,
                  eb0, eb1, vb0, vb1, enb0,
                  acc, s_is0, s_is1, s_id0, s_id1, s_c0, s_c1, s_db0, s_db1,
                  s_e0, s_e1, s_sc0, s_sc1, s_en0):
    c = lax.axis_index("c")
    s = lax.axis_index("s")
    isrc = (isrc0, isrc1)
    idst = (idst0, idst1)
    cbuf = (cbuf0, cbuf1)
    db = (db0, db1)
    eb = (eb0, eb1)
    vb = (vb0, vb1)
    s_is = (s_is0, s_is1)
    s_id = (s_id0, s_id1)
    s_c = (s_c0, s_c1)
    s_db = (s_db0, s_db1)
    s_e = (s_e0, s_e1)
    s_sc = (s_sc0, s_sc1)
    tb = s * EPT

    def zrow(rr, carry):
        for q in range(8):
            vb0[rr, pl.ds(q * 16, 16)] = jnp.zeros((16,), jnp.float32)
        return carry

    lax.fori_loop(0, CH, zrow, 0)
    nfull = NZ // CH
    for k in range(nfull):
        pltpu.sync_copy(vb0, acc.at[pl.ds(s * NZ + k * CH, CH)])
    rem = NZ - nfull * CH
    if rem:
        pltpu.sync_copy(vb0.at[pl.ds(0, rem)], acc.at[pl.ds(s * NZ + nfull * CH, rem)])

    @pl.when(s == 0)
    def _zero_tail():
        pltpu.sync_copy(vb0.at[pl.ds(0, NZT)], acc.at[pl.ds(NZ * NW, NZT)])

    plsc.subcore_barrier()

    def idx_issue(j, p):
        base = tb + j * CH
        pltpu.async_copy(srcadj.at[pl.ds(c * E + base, CH)], isrc[p], s_is[p])
        pltpu.async_copy(dstraw.at[pl.ds(base, CH)], idst[p], s_id[p])

    def idx_wait(p):
        pltpu.make_async_copy(srcadj.at[pl.ds(0, CH)], isrc[p], s_is[p]).wait()
        pltpu.make_async_copy(dstraw.at[pl.ds(0, CH)], idst[p], s_id[p]).wait()

    def cbuf_issue(j, p):
        base = c * E + tb + j * CH
        pltpu.async_copy(C.at[pl.ds(base, CH)], cbuf[p], s_c[p])

    def cbuf_wait(p):
        pltpu.make_async_copy(C.at[pl.ds(0, CH)], cbuf[p], s_c[p]).wait()

    def gathers_issue(p):
        pltpu.async_copy(DB.at[isrc[p]], db[p], s_db[p])
        pltpu.async_copy(Et.at[idst[p]], eb[p], s_e[p])

    def gathers_wait(p):
        pltpu.make_async_copy(DB.at[pl.ds(0, CH)], db[p], s_db[p]).wait()
        pltpu.make_async_copy(Et.at[pl.ds(0, CH)], eb[p], s_e[p]).wait()

    def scatter_issue(p):
        pltpu.async_copy(vb[p], acc.at[idst[p]], s_sc[p], add=True)

    def scatter_wait(p):
        pltpu.make_async_copy(DB.at[pl.ds(0, CH)], vb[p], s_sc[p]).wait()

    def en_issue(j):
        base = c * E + tb + j * CH
        pltpu.async_copy(enb0, en_hbm.at[pl.ds(base, CH)], s_en0)

    def en_wait():
        pltpu.make_async_copy(C.at[pl.ds(0, CH)], enb0, s_en0).wait()

    def compute(p):
        def row(rr, carry2):
            for q in range(4):
                ql = pl.ds(q * 16, 16)
                x = (cbuf[p][rr, ql] + db[p][rr, ql]
                     + eb[p][rr, pl.ds(c * HH + q * 16, 16)])
                sg = 1.0 / (1.0 + jnp.exp(-x))
                vb[p][rr, ql] = sg * db[p][rr, pl.ds(HH + q * 16, 16)]
                vb[p][rr, pl.ds(HH + q * 16, 16)] = sg
                if want_en:
                    enb0[rr, ql] = x
            return carry2

        lax.fori_loop(0, CH, row, 0, unroll=2)

    idx_issue(0, 0)
    idx_wait(0)
    gathers_issue(0)
    cbuf_issue(0, 0)
    idx_issue(1, 1)
    cbuf_wait(0)
    gathers_wait(0)
    compute(0)
    scatter_issue(0)
    if want_en:
        en_issue(0)
    idx_wait(1)
    gathers_issue(1)
    cbuf_issue(1, 1)
    scatter_wait(0)
    idx_issue(2, 0)
    cbuf_wait(1)
    gathers_wait(1)
    if want_en:
        en_wait()
    compute(1)
    scatter_issue(1)
    if want_en:
        en_issue(1)
    idx_wait(0)
    gathers_issue(0)
    cbuf_issue(2, 0)

    def pair(j2, carry):
        je = 2 * j2
        jo = je + 1

        scatter_wait(1)
        idx_issue(jo, 1)
        cbuf_wait(0)
        gathers_wait(0)
        if want_en:
            en_wait()
        compute(0)
        scatter_issue(0)
        if want_en:
            en_issue(je)
        idx_wait(1)
        gathers_issue(1)
        cbuf_issue(jo, 1)

        @pl.when(j2 < NPAIR - 1)
        def _pref_next():
            scatter_wait(0)
            idx_issue(jo + 1, 0)

        cbuf_wait(1)
        gathers_wait(1)
        if want_en:
            en_wait()
        compute(1)
        scatter_issue(1)
        if want_en:
            en_issue(jo)

        @pl.when(j2 < NPAIR - 1)
        def _pref_next2():
            idx_wait(0)
            gathers_issue(0)
            cbuf_issue(jo + 1, 0)

        return carry

    lax.fori_loop(1, NPAIR, pair, 0)

    scatter_wait(0)
    scatter_wait(1)
    if want_en:
        en_wait()
    plsc.subcore_barrier()
    pltpu.sync_copy(acc.at[pl.ds(s * NZ, NZ)], nd_hbm.at[c, pl.ds(s * NZ, NZ)])

    @pl.when(s == 0)
    def _flush_tail():
        pltpu.sync_copy(acc.at[pl.ds(NZ * NW, NZT)], nd_hbm.at[c, pl.ds(NZ * NW, NZT)])


def _make_sc_edge(want_en):
    outs = [
        jax.ShapeDtypeStruct((2, N, H), jnp.float32),
        jax.ShapeDtypeStruct((2 * E, HH) if want_en else (8, HH), jnp.float32),
    ]
    sems = [pltpu.SemaphoreType.DMA] * 13
    return pl.kernel(
        functools.partial(_sc_edge_body, want_en),
        out_type=tuple(outs),
        mesh=_SC_MESH,
        scratch_types=[
            pltpu.VMEM((CH,), jnp.int32),
            pltpu.VMEM((CH,), jnp.int32),
            pltpu.VMEM((CH,), jnp.int32),
            pltpu.VMEM((CH,), jnp.int32),
            pltpu.VMEM((CH, HH), jnp.float32),
            pltpu.VMEM((CH, HH), jnp.float32),
            pltpu.VMEM((CH, H), jnp.float32),
            pltpu.VMEM((CH, H), jnp.float32),
            pltpu.VMEM((CH, H), jnp.float32),
            pltpu.VMEM((CH, H), jnp.float32),
            pltpu.VMEM((CH, H), jnp.float32),
            pltpu.VMEM((CH, H), jnp.float32),
            pltpu.VMEM((CH, HH), jnp.float32),
            pltpu.VMEM_SHARED((N, H), jnp.float32),
        ] + sems,
    )


_sc_edge0 = _make_sc_edge(True)
_sc_edge1 = _make_sc_edge(False)


def _edge_stage_sc(C, DB, Et, srcadj, dstraw, want_en):
    fn = _sc_edge0 if want_en else _sc_edge1
    nd, en = fn(srcadj, dstraw, C.reshape(2 * E, HH),
                DB.reshape(2 * N, H), Et)
    if not want_en:
        return nd, None
    return nd, en.reshape(2, E, HH)


def _stats_body(en_ref, muvar_ref, acc_ref):
    i = pl.program_id(0)

    @pl.when(i == 0)
    def _init():
        acc_ref[...] = jnp.zeros((2, H), jnp.float32)

    blk = en_ref[...]
    s0 = jnp.sum(blk[0], axis=0)
    s1 = jnp.sum(blk[1], axis=0)
    q0 = jnp.sum(blk[0] * blk[0], axis=0)
    q1 = jnp.sum(blk[1] * blk[1], axis=0)
    acc_ref[0, :] += jnp.concatenate([s0, s1])
    acc_ref[1, :] += jnp.concatenate([q0, q1])

    @pl.when(i == (E // ER) - 1)
    def _fin():
        emu = acc_ref[0, :] / E
        muvar_ref[0, :] = emu
        muvar_ref[1, :] = acc_ref[1, :] / E - emu * emu


def _tc_stats(en):
    return pl.pallas_call(
        _stats_body,
        grid=(E // ER,),
        in_specs=[pl.BlockSpec((2, ER, HH), lambda i: (0, i, 0))],
        out_specs=pl.BlockSpec((2, H), lambda i: (0, 0)),
        out_shape=jax.ShapeDtypeStruct((2, H), jnp.float32),
        scratch_shapes=[pltpu.VMEM((2, H), jnp.float32)],
    )(en)


def _hupdate_body(h0_ref, A0_ref, nd_ref, gh_ref, bh_ref, h1_ref):
    num = jnp.concatenate([nd_ref[0][:, :HH], nd_ref[1][:, :HH]], axis=1)
    den = jnp.concatenate([nd_ref[0][:, HH:], nd_ref[1][:, HH:]], axis=1)
    h_new = A0_ref[...] + num / (den + 1e-6)
    mu = jnp.mean(h_new, axis=0, keepdims=True)
    var = jnp.mean((h_new - mu) ** 2, axis=0, keepdims=True)
    hbn = gh_ref[...] * (h_new - mu) / jnp.sqrt(var + 1e-5) + bh_ref[...]
    h1_ref[...] = h0_ref[...] + jnp.maximum(hbn, 0.0)


def _tc_hupdate(h0, A0, nd, gh, bh):
    return pl.pallas_call(
        _hupdate_body,
        out_shape=jax.ShapeDtypeStruct((N, H), jnp.float32),
    )(h0, A0, nd, gh, bh)


def _edge_mid_body(e_ref, en0_ref, muvar_ref, ge_ref, be_ref, Wembe_ref,
                   WC_ref, bC_ref, P_ref):
    en = jnp.concatenate([en0_ref[0], en0_ref[1]], axis=1)
    mu = muvar_ref[0][None, :]
    var = muvar_ref[1][None, :]
    t = ge_ref[...] * (en - mu) / jnp.sqrt(var + 1e-5) + be_ref[...]
    t = jnp.maximum(t, 0.0)
    F = jnp.dot(Wembe_ref[...], WC_ref[...], preferred_element_type=jnp.float32)
    p = (jnp.dot(t, WC_ref[...], preferred_element_type=jnp.float32)
         + jnp.dot(1.0 / e_ref[...], F, preferred_element_type=jnp.float32)
         + bC_ref[...])
    P_ref[0] = p[:, :HH]
    P_ref[1] = p[:, HH:]


def _tc_edge_mid(e, en0, muvar, ge, be, Wembe, WC1, bC1):
    return pl.pallas_call(
        _edge_mid_body,
        grid=(E // ER,),
        in_specs=[
            pl.BlockSpec((ER, DE), lambda i: (i, 0)),
            pl.BlockSpec((2, ER, HH), lambda i: (0, i, 0)),
            pl.BlockSpec((2, H), lambda i: (0, 0)),
            pl.BlockSpec((1, H), lambda i: (0, 0)),
            pl.BlockSpec((1, H), lambda i: (0, 0)),
            pl.BlockSpec((DE, H), lambda i: (0, 0)),
            pl.BlockSpec((H, H), lambda i: (0, 0)),
            pl.BlockSpec((1, H), lambda i: (0, 0)),
        ],
        out_specs=pl.BlockSpec((2, ER, HH), lambda i: (0, i, 0)),
        out_shape=jax.ShapeDtypeStruct((2, E, HH), jnp.float32),
    )(e, en0, muvar, ge, be, Wembe, WC1, bC1)


def _final_body(h1_ref, A1_ref, nd_ref, gh_ref, bh_ref, state_ref,
                action_ref, W1_ref, b1_ref, W2_ref, b2_ref, W3_ref, b3_ref,
                out_ref):
    num = jnp.concatenate([nd_ref[0][:, :HH], nd_ref[1][:, :HH]], axis=1)
    den = jnp.concatenate([nd_ref[0][:, HH:], nd_ref[1][:, HH:]], axis=1)
    h_new = A1_ref[...] + num / (den + 1e-6)
    mu = jnp.mean(h_new, axis=0, keepdims=True)
    var = jnp.mean((h_new - mu) ** 2, axis=0, keepdims=True)
    hbn = gh_ref[...] * (h_new - mu) / jnp.sqrt(var + 1e-5) + bh_ref[...]
    h2 = h1_ref[...] + jnp.maximum(hbn, 0.0)
    hg = jnp.mean(h2, axis=0, keepdims=True)
    z = jnp.concatenate([hg, state_ref[...], action_ref[...]], axis=1)
    z = jnp.maximum(jnp.dot(z, W1_ref[...], preferred_element_type=jnp.float32) + b1_ref[...], 0.0)
    z = jnp.maximum(jnp.dot(z, W2_ref[...], preferred_element_type=jnp.float32) + b2_ref[...], 0.0)
    out_ref[...] = jnp.dot(z, W3_ref[...], preferred_element_type=jnp.float32) + b3_ref[...]


def _tc_final(h1, A1, nd, gh, bh, state, action, W1, b1, W2, b2, W3, b3):
    return pl.pallas_call(
        _final_body,
        out_shape=jax.ShapeDtypeStruct((1, 1), jnp.float32),
    )(h1, A1, nd, gh, bh, state, action, W1, b1, W2, b2, W3, b3)


def kernel(h, e, edge_index, state, action, W_emb_h, W_emb_e, WA, bA, WB, bB,
           WC, bC, WD, bD, WE, bE, gh, bh, ge, be, W1, b1, W2, b2, W3, b3):
    r = lambda v: v.reshape(1, -1)

    src = edge_index[0]
    dst = edge_index[1]
    srcadj = jnp.concatenate([src, src + N])

    h0, A0, DB0, Et0 = _tc_tables(h, W_emb_h, WA[0], r(bA[0]), WB[0], r(bB[0]),
                                  WD[0], r(bD[0]), WE[0], r(bE[0]), True)
    C0 = _tc_edge_pre(e, W_emb_e, WC[0], r(bC[0]))
    nd0, en0 = _edge_stage_sc(C0, DB0, Et0, srcadj, dst, True)

    muvar0 = _tc_stats(en0)
    h1 = _tc_hupdate(h0, A0, nd0, r(gh[0]), r(bh[0]))
    _, A1, DB1, Et1 = _tc_tables(h1, W_emb_h, WA[1], r(bA[1]), WB[1], r(bB[1]),
                                 WD[1], r(bD[1]), WE[1], r(bE[1]), False)
    P1 = _tc_edge_mid(e, en0, muvar0, r(ge[0]), r(be[0]), W_emb_e, WC[1], r(bC[1]))

    nd1, _ = _edge_stage_sc(P1, DB1, Et1, srcadj, dst, False)

    return _tc_final(h1, A1, nd1, r(gh[1]), r(bh[1]), state, action,
                     W1, r(b1), W2, r(b2), W3, r(b3))
